# baseline (speedup 1.0000x reference)
import functools

import jax
import jax.numpy as jnp
from jax import lax
from jax.experimental import pallas as pl
from jax.experimental.pallas import tpu as pltpu
from jax.experimental.pallas import tpu_sc as plsc

N = 10000
D = 128
E = 320000
NT = 5
NC, NS = 2, 16
NW = NC * NS
NPAD = 10240
ROWS_PER_TILE = NPAD // NS
EDGES_PER_TILE = E // NW
EPAD = NW * 10240
EDGES_PER_TILE_P = EPAD // NW
CHUNK = 200
NCHUNK = EDGES_PER_TILE // CHUNK
ACHUNK = 128
ANCHUNK = EDGES_PER_TILE_P // ACHUNK
IDXG = 40
NGROUP = ANCHUNK // IDXG

_sc_mesh = plsc.VectorSubcoreMesh(core_axis_name="c", subcore_axis_name="s")


@functools.partial(
    pl.kernel,
    out_type=jax.ShapeDtypeStruct((NC * NPAD,), jnp.float32),
    mesh=_sc_mesh,
    scratch_types=[
        pltpu.VMEM((EDGES_PER_TILE,), jnp.int32),
        pltpu.VMEM((EDGES_PER_TILE,), jnp.float32),
        pltpu.VMEM_SHARED((NPAD,), jnp.float32),
    ],
)
def _deg_kernel(dst_hbm, ones_hbm, init_hbm, out_hbm, idx_v, ones_v, deg_sh):
    cid = lax.axis_index("c")
    sid = lax.axis_index("s")
    wid = cid * NS + sid
    row0 = sid * ROWS_PER_TILE
    pltpu.sync_copy(init_hbm.at[pl.ds(cid * NPAD + row0, ROWS_PER_TILE)],
                    deg_sh.at[pl.ds(row0, ROWS_PER_TILE)])
    pltpu.sync_copy(dst_hbm.at[pl.ds(wid * EDGES_PER_TILE, EDGES_PER_TILE)], idx_v)
    pltpu.sync_copy(ones_hbm, ones_v)
    plsc.subcore_barrier()
    pltpu.sync_copy(ones_v, deg_sh.at[idx_v], add=True)
    plsc.subcore_barrier()
    pltpu.sync_copy(deg_sh.at[pl.ds(row0, ROWS_PER_TILE)],
                    out_hbm.at[pl.ds(cid * NPAD + row0, ROWS_PER_TILE)])


def _prep_body(degp_ref, z_ref, wg_ref, y_ref, dinv_ref, w_ref, loss_ref):
    deg = degp_ref[:, 0:1] + degp_ref[:, 1:2]
    dinv = lax.rsqrt(jnp.maximum(deg, 1.0))
    z = z_ref[...]
    y_ref[...] = dinv * z
    dinv_ref[...] = dinv
    logits = jnp.dot(z, wg_ref[...], preferred_element_type=jnp.float32)
    iota = lax.broadcasted_iota(jnp.int32, (N, NT), 1)
    m1 = jnp.max(logits, axis=1, keepdims=True)
    i1 = jnp.min(jnp.where(logits == m1, iota, NT), axis=1, keepdims=True)
    masked = jnp.where(iota == i1, -jnp.inf, logits)
    m2 = jnp.max(masked, axis=1, keepdims=True)
    i2 = jnp.min(jnp.where(masked == m2, iota, NT), axis=1, keepdims=True)
    e = jnp.exp(m2 - m1)
    w1 = 1.0 / (1.0 + e)
    w2 = e / (1.0 + e)
    w = jnp.where(iota == i1, w1, 0.0) + jnp.where(iota == i2, w2, 0.0)
    w_ref[...] = w
    imp = jnp.sum(w, axis=0)
    mean = jnp.mean(imp)
    std = jnp.sqrt(jnp.mean((imp - mean) ** 2))
    loss_ref[...] = jnp.reshape((std / (mean + 1e-9)) ** 2, (1, 1))


def _prep(degp_t, z, w_gate):
    return pl.pallas_call(
        _prep_body,
        out_shape=(jax.ShapeDtypeStruct((N, D), jnp.float32),
                   jax.ShapeDtypeStruct((N, 1), jnp.float32),
                   jax.ShapeDtypeStruct((N, NT), jnp.float32),
                   jax.ShapeDtypeStruct((1, 1), jnp.float32)),
    )(degp_t, z, w_gate)


@functools.partial(
    pl.kernel,
    out_type=jax.ShapeDtypeStruct((NC, NPAD, D), jnp.float32),
    mesh=_sc_mesh,
    scratch_types=[
        pltpu.VMEM((IDXG, ACHUNK), jnp.int32),
        pltpu.VMEM((IDXG, ACHUNK), jnp.int32),
        pltpu.VMEM((ACHUNK, D), jnp.float32),
        pltpu.VMEM((ACHUNK, D), jnp.float32),
        pltpu.VMEM_SHARED((NPAD, D), jnp.float32),
        pltpu.SemaphoreType.DMA,
        pltpu.SemaphoreType.DMA,
        pltpu.SemaphoreType.DMA,
        pltpu.SemaphoreType.DMA,
    ],
)
def _acc_kernel(src_hbm, dst_hbm, y_hbm, zeros_hbm, out_hbm,
                srcs_v, dsts_v, rows0, rows1, acc_sh, sem0, sem1, ssem0, ssem1):
    cid = lax.axis_index("c")
    sid = lax.axis_index("s")
    wid = cid * NS + sid
    row0 = sid * ROWS_PER_TILE
    pltpu.sync_copy(zeros_hbm, acc_sh.at[pl.ds(row0, ROWS_PER_TILE)])
    plsc.subcore_barrier()

    dummy = y_hbm.at[pl.ds(0, ACHUNK)]

    for g in range(NGROUP):
        pltpu.sync_copy(src_hbm.at[wid, pl.ds(g * IDXG, IDXG)], srcs_v)
        pltpu.sync_copy(dst_hbm.at[wid, pl.ds(g * IDXG, IDXG)], dsts_v)

        pltpu.async_copy(y_hbm.at[srcs_v.at[0]], rows0, sem0)
        pltpu.async_copy(y_hbm.at[srcs_v.at[1]], rows1, sem1)

        sdummy = acc_sh.at[pl.ds(0, ACHUNK)]

        def body(j, _):
            pltpu.make_async_copy(dummy, rows0, sem0).wait()
            pltpu.async_copy(rows0, acc_sh.at[dsts_v.at[2 * j]], ssem0,
                             add=True)
            pltpu.make_async_copy(dummy, rows1, sem1).wait()
            pltpu.async_copy(rows1, acc_sh.at[dsts_v.at[2 * j + 1]], ssem1,
                             add=True)
            pltpu.make_async_copy(rows0, sdummy, ssem0).wait()
            pltpu.async_copy(y_hbm.at[srcs_v.at[2 * j + 2]], rows0, sem0)
            pltpu.make_async_copy(rows1, sdummy, ssem1).wait()
            pltpu.async_copy(y_hbm.at[srcs_v.at[2 * j + 3]], rows1, sem1)
            return 0

        lax.fori_loop(0, IDXG // 2 - 1, body, 0)
        pltpu.make_async_copy(dummy, rows0, sem0).wait()
        pltpu.sync_copy(rows0, acc_sh.at[dsts_v.at[IDXG - 2]], add=True)
        pltpu.make_async_copy(dummy, rows1, sem1).wait()
        pltpu.sync_copy(rows1, acc_sh.at[dsts_v.at[IDXG - 1]], add=True)

    plsc.subcore_barrier()
    pltpu.sync_copy(acc_sh.at[pl.ds(row0, ROWS_PER_TILE)],
                    out_hbm.at[cid, pl.ds(row0, ROWS_PER_TILE)])


RB = 1000


def _fuse_body(acc_ref, y_ref, dinv_ref, w_ref, wf_ref, out_ref):
    a = acc_ref[0] + acc_ref[1]
    h = dinv_ref[...] * (a + y_ref[...])
    hw = jnp.dot(h, wf_ref[...], preferred_element_type=jnp.float32)
    out = w_ref[:, 0:1] * hw[:, 0:D]
    for t in range(1, NT):
        out += w_ref[:, t:t + 1] * hw[:, t * D:(t + 1) * D]
    out_ref[...] = out


def _fuse(accp, y, dinv, w, w_flat):
    return pl.pallas_call(
        _fuse_body,
        grid=(N // RB,),
        in_specs=[
            pl.BlockSpec((NC, RB, D), lambda i: (0, i, 0)),
            pl.BlockSpec((RB, D), lambda i: (i, 0)),
            pl.BlockSpec((RB, 1), lambda i: (i, 0)),
            pl.BlockSpec((RB, NT), lambda i: (i, 0)),
            pl.BlockSpec((D, NT * D), lambda i: (0, 0)),
        ],
        out_specs=pl.BlockSpec((RB, D), lambda i: (i, 0)),
        out_shape=jax.ShapeDtypeStruct((N, D), jnp.float32),
    )(accp, y, dinv, w, w_flat)


def kernel(z, edge_index, W_gate, W_experts):
    src = edge_index[0].astype(jnp.int32)
    dst = edge_index[1].astype(jnp.int32)
    ones = jnp.ones((EDGES_PER_TILE,), jnp.float32)
    init = jnp.concatenate([jnp.ones((NPAD,), jnp.float32),
                            jnp.zeros((NPAD,), jnp.float32)])
    zeros = jnp.zeros((ROWS_PER_TILE, D), jnp.float32)
    w_flat = W_experts.transpose(1, 0, 2).reshape(D, NT * D)

    npadedge = EPAD - E
    src_p = jnp.concatenate([src, jnp.arange(npadedge, dtype=jnp.int32) % N])
    dst_p = jnp.concatenate([dst, N + jnp.arange(npadedge, dtype=jnp.int32)
                             % (NPAD - N)])
    src3d = src_p.reshape(NW, ANCHUNK, ACHUNK)
    dst3d = dst_p.reshape(NW, ANCHUNK, ACHUNK)

    degp = _deg_kernel(dst, ones, init)
    degp_t = jnp.stack([degp[:N], degp[NPAD:NPAD + N]], axis=1)
    y, dinv, w_dense, loss = _prep(degp_t, z, W_gate)
    accp = _acc_kernel(src3d, dst3d, y, zeros)
    fusion = _fuse(accp, y, dinv, w_dense, w_flat)
    return fusion, loss[0, 0], w_dense


# speedup vs baseline: 1.2369x; 1.2369x over previous
import functools

import jax
import jax.numpy as jnp
from jax import lax
from jax.experimental import pallas as pl
from jax.experimental.pallas import tpu as pltpu
from jax.experimental.pallas import tpu_sc as plsc

N = 10000
D = 128
E = 320000
NT = 5
NC, NS = 2, 16
NW = NC * NS
NPAD = 10240
ROWS_PER_TILE = NPAD // NS
EDGES_PER_TILE = E // NW
EPAD = NW * 10240
EDGES_PER_TILE_P = EPAD // NW
CHUNK = 200
NCHUNK = EDGES_PER_TILE // CHUNK
ACHUNK = 128
ANCHUNK = EDGES_PER_TILE_P // ACHUNK
IDXG = 40
NGROUP = ANCHUNK // IDXG

_sc_mesh = plsc.VectorSubcoreMesh(core_axis_name="c", subcore_axis_name="s")


@functools.partial(
    pl.kernel,
    out_type=jax.ShapeDtypeStruct((NC * NPAD,), jnp.float32),
    mesh=_sc_mesh,
    scratch_types=[
        pltpu.VMEM((EDGES_PER_TILE,), jnp.int32),
        pltpu.VMEM((EDGES_PER_TILE,), jnp.float32),
        pltpu.VMEM_SHARED((NPAD,), jnp.float32),
    ],
)
def _deg_kernel(dst_hbm, ones_hbm, init_hbm, out_hbm, idx_v, ones_v, deg_sh):
    cid = lax.axis_index("c")
    sid = lax.axis_index("s")
    wid = cid * NS + sid
    row0 = sid * ROWS_PER_TILE
    pltpu.sync_copy(init_hbm.at[pl.ds(cid * NPAD + row0, ROWS_PER_TILE)],
                    deg_sh.at[pl.ds(row0, ROWS_PER_TILE)])
    pltpu.sync_copy(dst_hbm.at[pl.ds(wid * EDGES_PER_TILE, EDGES_PER_TILE)], idx_v)
    pltpu.sync_copy(ones_hbm, ones_v)
    plsc.subcore_barrier()
    pltpu.sync_copy(ones_v, deg_sh.at[idx_v], add=True)
    plsc.subcore_barrier()
    pltpu.sync_copy(deg_sh.at[pl.ds(row0, ROWS_PER_TILE)],
                    out_hbm.at[pl.ds(cid * NPAD + row0, ROWS_PER_TILE)])


def _prep_body(degp_ref, z_ref, wg_ref, y_ref, dinv_ref, w_ref, loss_ref):
    deg = degp_ref[:, 0:1] + degp_ref[:, 1:2]
    dinv = lax.rsqrt(jnp.maximum(deg, 1.0))
    z = z_ref[...]
    y_ref[...] = dinv * z
    dinv_ref[...] = dinv
    logits = lax.dot_general(wg_ref[...], z, (((0,), (1,)), ((), ())),
                             preferred_element_type=jnp.float32)
    iota = lax.broadcasted_iota(jnp.int32, (NT, N), 0)
    m1 = jnp.max(logits, axis=0, keepdims=True)
    i1 = jnp.min(jnp.where(logits == m1, iota, NT), axis=0, keepdims=True)
    masked = jnp.where(iota == i1, -jnp.inf, logits)
    m2 = jnp.max(masked, axis=0, keepdims=True)
    i2 = jnp.min(jnp.where(masked == m2, iota, NT), axis=0, keepdims=True)
    e = jnp.exp(m2 - m1)
    w1 = 1.0 / (1.0 + e)
    w2 = e / (1.0 + e)
    w = jnp.where(iota == i1, w1, 0.0) + jnp.where(iota == i2, w2, 0.0)
    w_ref[...] = w.T
    imp = jnp.sum(w, axis=1)
    mean = jnp.mean(imp)
    std = jnp.sqrt(jnp.mean((imp - mean) ** 2))
    loss_ref[...] = jnp.reshape((std / (mean + 1e-9)) ** 2, (1, 1))


def _prep(degp_t, z, w_gate):
    return pl.pallas_call(
        _prep_body,
        out_shape=(jax.ShapeDtypeStruct((N, D), jnp.float32),
                   jax.ShapeDtypeStruct((N, 1), jnp.float32),
                   jax.ShapeDtypeStruct((N, NT), jnp.float32),
                   jax.ShapeDtypeStruct((1, 1), jnp.float32)),
    )(degp_t, z, w_gate)


@functools.partial(
    pl.kernel,
    out_type=jax.ShapeDtypeStruct((NC, NPAD, D), jnp.float32),
    mesh=_sc_mesh,
    scratch_types=[
        pltpu.VMEM((IDXG, ACHUNK), jnp.int32),
        pltpu.VMEM((IDXG, ACHUNK), jnp.int32),
        pltpu.VMEM((ACHUNK, D), jnp.float32),
        pltpu.VMEM((ACHUNK, D), jnp.float32),
        pltpu.VMEM_SHARED((NPAD, D), jnp.float32),
        pltpu.SemaphoreType.DMA,
        pltpu.SemaphoreType.DMA,
    ],
)
def _acc_kernel(src_hbm, dst_hbm, y_hbm, zeros_hbm, out_hbm,
                srcs_v, dsts_v, rows0, rows1, acc_sh, sem0, sem1):
    cid = lax.axis_index("c")
    sid = lax.axis_index("s")
    wid = cid * NS + sid
    row0 = sid * ROWS_PER_TILE
    pltpu.sync_copy(zeros_hbm, acc_sh.at[pl.ds(row0, ROWS_PER_TILE)])
    plsc.subcore_barrier()

    dummy = y_hbm.at[pl.ds(0, ACHUNK)]

    for g in range(NGROUP):
        pltpu.sync_copy(src_hbm.at[wid, pl.ds(g * IDXG, IDXG)], srcs_v)
        pltpu.sync_copy(dst_hbm.at[wid, pl.ds(g * IDXG, IDXG)], dsts_v)

        pltpu.async_copy(y_hbm.at[srcs_v.at[0]], rows0, sem0)
        pltpu.async_copy(y_hbm.at[srcs_v.at[1]], rows1, sem1)

        def body(j, _):
            pltpu.make_async_copy(dummy, rows0, sem0).wait()
            pltpu.sync_copy(rows0, acc_sh.at[dsts_v.at[2 * j]], add=True)
            pltpu.async_copy(y_hbm.at[srcs_v.at[2 * j + 2]], rows0, sem0)
            pltpu.make_async_copy(dummy, rows1, sem1).wait()
            pltpu.sync_copy(rows1, acc_sh.at[dsts_v.at[2 * j + 1]], add=True)
            pltpu.async_copy(y_hbm.at[srcs_v.at[2 * j + 3]], rows1, sem1)
            return 0

        lax.fori_loop(0, IDXG // 2 - 1, body, 0)
        pltpu.make_async_copy(dummy, rows0, sem0).wait()
        pltpu.sync_copy(rows0, acc_sh.at[dsts_v.at[IDXG - 2]], add=True)
        pltpu.make_async_copy(dummy, rows1, sem1).wait()
        pltpu.sync_copy(rows1, acc_sh.at[dsts_v.at[IDXG - 1]], add=True)

    plsc.subcore_barrier()
    pltpu.sync_copy(acc_sh.at[pl.ds(row0, ROWS_PER_TILE)],
                    out_hbm.at[cid, pl.ds(row0, ROWS_PER_TILE)])


RB = 1000


def _fuse_body(acc_ref, y_ref, dinv_ref, w_ref, wf_ref, out_ref):
    a = acc_ref[0] + acc_ref[1]
    h = dinv_ref[...] * (a + y_ref[...])
    hw = jnp.dot(h, wf_ref[...], preferred_element_type=jnp.float32)
    out = w_ref[:, 0:1] * hw[:, 0:D]
    for t in range(1, NT):
        out += w_ref[:, t:t + 1] * hw[:, t * D:(t + 1) * D]
    out_ref[...] = out


def _fuse(accp, y, dinv, w, w_flat):
    return pl.pallas_call(
        _fuse_body,
        grid=(N // RB,),
        in_specs=[
            pl.BlockSpec((NC, RB, D), lambda i: (0, i, 0)),
            pl.BlockSpec((RB, D), lambda i: (i, 0)),
            pl.BlockSpec((RB, 1), lambda i: (i, 0)),
            pl.BlockSpec((RB, NT), lambda i: (i, 0)),
            pl.BlockSpec((D, NT * D), lambda i: (0, 0)),
        ],
        out_specs=pl.BlockSpec((RB, D), lambda i: (i, 0)),
        out_shape=jax.ShapeDtypeStruct((N, D), jnp.float32),
    )(accp, y, dinv, w, w_flat)


def kernel(z, edge_index, W_gate, W_experts):
    src = edge_index[0].astype(jnp.int32)
    dst = edge_index[1].astype(jnp.int32)
    ones = jnp.ones((EDGES_PER_TILE,), jnp.float32)
    init = jnp.concatenate([jnp.ones((NPAD,), jnp.float32),
                            jnp.zeros((NPAD,), jnp.float32)])
    zeros = jnp.zeros((ROWS_PER_TILE, D), jnp.float32)
    w_flat = W_experts.transpose(1, 0, 2).reshape(D, NT * D)

    npadedge = EPAD - E
    src_p = jnp.concatenate([src, jnp.arange(npadedge, dtype=jnp.int32) % N])
    dst_p = jnp.concatenate([dst, N + jnp.arange(npadedge, dtype=jnp.int32)
                             % (NPAD - N)])
    src3d = src_p.reshape(NW, ANCHUNK, ACHUNK)
    dst3d = dst_p.reshape(NW, ANCHUNK, ACHUNK)

    degp = _deg_kernel(dst, ones, init)
    degp_t = jnp.stack([degp[:N], degp[NPAD:NPAD + N]], axis=1)
    y, dinv, w_dense, loss = _prep(degp_t, z, W_gate)
    accp = _acc_kernel(src3d, dst3d, y, zeros)
    fusion = _fuse(accp, y, dinv, w_dense, w_flat)
    return fusion, loss[0, 0], w_dense


# speedup vs baseline: 1.2567x; 1.0160x over previous
import functools

import jax
import jax.numpy as jnp
from jax import lax
from jax.experimental import pallas as pl
from jax.experimental.pallas import tpu as pltpu
from jax.experimental.pallas import tpu_sc as plsc

N = 10000
D = 128
E = 320000
NT = 5
NC, NS = 2, 16
NW = NC * NS
NPAD = 10240
ROWS_PER_TILE = NPAD // NS
EDGES_PER_TILE = E // NW
EPAD = NW * 10240
EDGES_PER_TILE_P = EPAD // NW
CHUNK = 200
NCHUNK = EDGES_PER_TILE // CHUNK
ACHUNK = 128
ANCHUNK = EDGES_PER_TILE_P // ACHUNK
IDXG = 40
NGROUP = ANCHUNK // IDXG

_sc_mesh = plsc.VectorSubcoreMesh(core_axis_name="c", subcore_axis_name="s")


@functools.partial(
    pl.kernel,
    out_type=jax.ShapeDtypeStruct((NC * NPAD,), jnp.float32),
    mesh=_sc_mesh,
    scratch_types=[
        pltpu.VMEM((EDGES_PER_TILE,), jnp.int32),
        pltpu.VMEM((EDGES_PER_TILE,), jnp.float32),
        pltpu.VMEM_SHARED((NPAD,), jnp.float32),
    ],
)
def _deg_kernel(dst_hbm, ones_hbm, init_hbm, out_hbm, idx_v, ones_v, deg_sh):
    cid = lax.axis_index("c")
    sid = lax.axis_index("s")
    wid = cid * NS + sid
    row0 = sid * ROWS_PER_TILE
    pltpu.sync_copy(init_hbm.at[pl.ds(cid * NPAD + row0, ROWS_PER_TILE)],
                    deg_sh.at[pl.ds(row0, ROWS_PER_TILE)])
    pltpu.sync_copy(dst_hbm.at[pl.ds(wid * EDGES_PER_TILE, EDGES_PER_TILE)], idx_v)
    pltpu.sync_copy(ones_hbm, ones_v)
    plsc.subcore_barrier()
    pltpu.sync_copy(ones_v, deg_sh.at[idx_v], add=True)
    plsc.subcore_barrier()
    pltpu.sync_copy(deg_sh.at[pl.ds(row0, ROWS_PER_TILE)],
                    out_hbm.at[pl.ds(cid * NPAD + row0, ROWS_PER_TILE)])


def _prep_body(degp_ref, z_ref, wg_ref, y_ref, dinv_ref, w_ref, loss_ref):
    deg = degp_ref[:, 0:1] + degp_ref[:, 1:2]
    dinv = lax.rsqrt(jnp.maximum(deg, 1.0))
    z = z_ref[...]
    y_ref[...] = dinv * z
    dinv_ref[...] = dinv
    logits = lax.dot_general(wg_ref[...], z, (((0,), (1,)), ((), ())),
                             preferred_element_type=jnp.float32)
    iota = lax.broadcasted_iota(jnp.int32, (NT, N), 0)
    m1 = jnp.max(logits, axis=0, keepdims=True)
    i1 = jnp.min(jnp.where(logits == m1, iota, NT), axis=0, keepdims=True)
    masked = jnp.where(iota == i1, -jnp.inf, logits)
    m2 = jnp.max(masked, axis=0, keepdims=True)
    i2 = jnp.min(jnp.where(masked == m2, iota, NT), axis=0, keepdims=True)
    e = jnp.exp(m2 - m1)
    w1 = 1.0 / (1.0 + e)
    w2 = e / (1.0 + e)
    w = jnp.where(iota == i1, w1, 0.0) + jnp.where(iota == i2, w2, 0.0)
    w_ref[...] = w.T
    imp = jnp.sum(w, axis=1)
    mean = jnp.mean(imp)
    std = jnp.sqrt(jnp.mean((imp - mean) ** 2))
    loss_ref[...] = jnp.reshape((std / (mean + 1e-9)) ** 2, (1, 1))


def _prep(degp_t, z, w_gate):
    return pl.pallas_call(
        _prep_body,
        out_shape=(jax.ShapeDtypeStruct((N, D), jnp.float32),
                   jax.ShapeDtypeStruct((N, 1), jnp.float32),
                   jax.ShapeDtypeStruct((N, NT), jnp.float32),
                   jax.ShapeDtypeStruct((1, 1), jnp.float32)),
    )(degp_t, z, w_gate)


@functools.partial(
    pl.kernel,
    out_type=jax.ShapeDtypeStruct((NC, NPAD, D), jnp.float32),
    mesh=_sc_mesh,
    scratch_types=[
        pltpu.VMEM((IDXG, ACHUNK), jnp.int32),
        pltpu.VMEM((IDXG, ACHUNK), jnp.int32),
        pltpu.VMEM((ACHUNK, D), jnp.float32),
        pltpu.VMEM((ACHUNK, D), jnp.float32),
        pltpu.VMEM_SHARED((NPAD, D), jnp.float32),
        pltpu.SemaphoreType.DMA,
        pltpu.SemaphoreType.DMA,
    ],
)
def _acc_kernel(src_hbm, dst_hbm, y_hbm, zeros_hbm, out_hbm,
                srcs_v, dsts_v, rows0, rows1, acc_sh, sem0, sem1):
    cid = lax.axis_index("c")
    sid = lax.axis_index("s")
    wid = cid * NS + sid
    row0 = sid * ROWS_PER_TILE
    zcp = pltpu.async_copy(zeros_hbm, acc_sh.at[pl.ds(row0, ROWS_PER_TILE)],
                           sem0)

    dummy = y_hbm.at[pl.ds(0, ACHUNK)]

    for g in range(NGROUP):
        pltpu.sync_copy(src_hbm.at[wid, pl.ds(g * IDXG, IDXG)], srcs_v)
        pltpu.sync_copy(dst_hbm.at[wid, pl.ds(g * IDXG, IDXG)], dsts_v)
        if g == 0:
            zcp.wait()
            plsc.subcore_barrier()

        pltpu.async_copy(y_hbm.at[srcs_v.at[0]], rows0, sem0)
        pltpu.async_copy(y_hbm.at[srcs_v.at[1]], rows1, sem1)

        def body(j, _):
            pltpu.make_async_copy(dummy, rows0, sem0).wait()
            pltpu.sync_copy(rows0, acc_sh.at[dsts_v.at[2 * j]], add=True)
            pltpu.async_copy(y_hbm.at[srcs_v.at[2 * j + 2]], rows0, sem0)
            pltpu.make_async_copy(dummy, rows1, sem1).wait()
            pltpu.sync_copy(rows1, acc_sh.at[dsts_v.at[2 * j + 1]], add=True)
            pltpu.async_copy(y_hbm.at[srcs_v.at[2 * j + 3]], rows1, sem1)
            return 0

        lax.fori_loop(0, IDXG // 2 - 1, body, 0)
        pltpu.make_async_copy(dummy, rows0, sem0).wait()
        pltpu.sync_copy(rows0, acc_sh.at[dsts_v.at[IDXG - 2]], add=True)
        pltpu.make_async_copy(dummy, rows1, sem1).wait()
        pltpu.sync_copy(rows1, acc_sh.at[dsts_v.at[IDXG - 1]], add=True)

    plsc.subcore_barrier()
    pltpu.sync_copy(acc_sh.at[pl.ds(row0, ROWS_PER_TILE)],
                    out_hbm.at[cid, pl.ds(row0, ROWS_PER_TILE)])


RB = 2000


def _fuse_body(acc_ref, y_ref, dinv_ref, w_ref, wf_ref, out_ref):
    a = acc_ref[0] + acc_ref[1]
    h = dinv_ref[...] * (a + y_ref[...])
    hw = jnp.dot(h, wf_ref[...], preferred_element_type=jnp.float32)
    out = w_ref[:, 0:1] * hw[:, 0:D]
    for t in range(1, NT):
        out += w_ref[:, t:t + 1] * hw[:, t * D:(t + 1) * D]
    out_ref[...] = out


def _fuse(accp, y, dinv, w, w_flat):
    return pl.pallas_call(
        _fuse_body,
        grid=(N // RB,),
        in_specs=[
            pl.BlockSpec((NC, RB, D), lambda i: (0, i, 0)),
            pl.BlockSpec((RB, D), lambda i: (i, 0)),
            pl.BlockSpec((RB, 1), lambda i: (i, 0)),
            pl.BlockSpec((RB, NT), lambda i: (i, 0)),
            pl.BlockSpec((D, NT * D), lambda i: (0, 0)),
        ],
        out_specs=pl.BlockSpec((RB, D), lambda i: (i, 0)),
        out_shape=jax.ShapeDtypeStruct((N, D), jnp.float32),
    )(accp, y, dinv, w, w_flat)


def kernel(z, edge_index, W_gate, W_experts):
    src = edge_index[0].astype(jnp.int32)
    dst = edge_index[1].astype(jnp.int32)
    ones = jnp.ones((EDGES_PER_TILE,), jnp.float32)
    init = jnp.concatenate([jnp.ones((NPAD,), jnp.float32),
                            jnp.zeros((NPAD,), jnp.float32)])
    zeros = jnp.zeros((ROWS_PER_TILE, D), jnp.float32)
    w_flat = W_experts.transpose(1, 0, 2).reshape(D, NT * D)

    npadedge = EPAD - E
    src_p = jnp.concatenate([src, jnp.arange(npadedge, dtype=jnp.int32) % N])
    dst_p = jnp.concatenate([dst, N + jnp.arange(npadedge, dtype=jnp.int32)
                             % (NPAD - N)])
    src3d = src_p.reshape(NW, ANCHUNK, ACHUNK)
    dst3d = dst_p.reshape(NW, ANCHUNK, ACHUNK)

    degp = _deg_kernel(dst, ones, init)
    degp_t = jnp.stack([degp[:N], degp[NPAD:NPAD + N]], axis=1)
    y, dinv, w_dense, loss = _prep(degp_t, z, W_gate)
    accp = _acc_kernel(src3d, dst3d, y, zeros)
    fusion = _fuse(accp, y, dinv, w_dense, w_flat)
    return fusion, loss[0, 0], w_dense
